# baseline (speedup 1.0000x reference)
import functools

import jax
import jax.numpy as jnp
from jax import lax
from jax.experimental import pallas as pl
from jax.experimental.pallas import tpu as pltpu
from jax.experimental.pallas import tpu_sc as plsc

N_NODES = 10000
NPAD = 10240
NPAD1 = 10496
DUMP = NPAD
E = 160000
E_PAD = 163840
NSUB = 16
NCORE = 2
CK = 128
SL = 128
MT = 256
NB = NPAD // MT
NB1 = NPAD1 // MT
STRIPE = NPAD1 // NSUB
AMP = 0.1
LEVELS = [[3, 4], [1, 2], [0, 1]]
VOL_SIZES = (64, 32, 16, 8, 4)
VOL_CH = (16, 32, 64, 128, 256)
MAXSLAB = 3


@functools.lru_cache(maxsize=None)
def _mesh_sc():
    return plsc.VectorSubcoreMesh(core_axis_name="c", subcore_axis_name="s")


def _prep_body(ei_ref, gsrc_ref, sdst_ref, sdeg_ref):
    src = ei_ref[0, :]
    dst = ei_ref[1, :]
    nonself = src != dst
    sdst_ref[...] = jnp.where(nonself, dst, DUMP)
    sdeg_ref[...] = jnp.where(nonself, src, DUMP)
    gsrc_ref[...] = jnp.concatenate(
        [src + k * NPAD for k in range(MAXSLAB)], axis=0)


def _prep(ei_pad):
    return pl.pallas_call(
        _prep_body,
        out_shape=(
            jax.ShapeDtypeStruct((MAXSLAB * E_PAD,), jnp.int32),
            jax.ShapeDtypeStruct((E_PAD,), jnp.int32),
            jax.ShapeDtypeStruct((E_PAD,), jnp.int32),
        ),
    )(ei_pad)


@functools.lru_cache(maxsize=None)
def _make_deg():
    @functools.partial(
        pl.kernel,
        out_type=jax.ShapeDtypeStruct((2 * NPAD1, SL), jnp.float32),
        mesh=_mesh_sc(),
        scratch_types=[
            pltpu.VMEM((E_PAD // (NCORE * NSUB) // CK, CK), jnp.int32),
            pltpu.VMEM((CK, SL), jnp.float32),
            pltpu.VMEM_SHARED((NPAD1, SL), jnp.float32),
            pltpu.SemaphoreType.DMA,
            pltpu.SemaphoreType.DMA,
            pltpu.SemaphoreType.DMA,
            pltpu.SemaphoreType.DMA,
        ],
    )
    def _deg_sc(zeros_hbm, sdeg_hbm, out_hbm, si_v, ones_v, acc_ref,
                s0, s1, s2, s3):
        c = lax.axis_index("c")
        s = lax.axis_index("s")

        def obody(r, _):
            for j in range(SL // 16):
                ones_v[r, pl.ds(j * 16, 16)] = jnp.ones((16,), jnp.float32)
            return 0

        lax.fori_loop(0, CK, obody, 0)
        base = s * STRIPE
        nrow = E_PAD // (NCORE * NSUB) // CK
        rbase = c * (E_PAD // 2 // CK) + s * nrow
        pltpu.sync_copy(sdeg_hbm.at[pl.ds(rbase, nrow)], si_v)
        pltpu.sync_copy(
            zeros_hbm.at[pl.ds(pl.multiple_of(base, 8), STRIPE)],
            acc_ref.at[pl.ds(pl.multiple_of(base, 8), STRIPE)])
        plsc.subcore_barrier()
        sems = (s0, s1, s2, s3)

        def ebody(k, _):
            for b in range(4):
                pltpu.async_copy(ones_v, acc_ref.at[si_v.at[4 * k + b]],
                                 sems[b], add=True)
            for b in range(4):
                pltpu.make_async_copy(out_hbm.at[pl.ds(0, CK)], ones_v,
                                      sems[b]).wait()
            return 0

        lax.fori_loop(0, E_PAD // (NCORE * NSUB) // CK // 4, ebody, 0)
        plsc.subcore_barrier()
        pltpu.sync_copy(
            acc_ref.at[pl.ds(base, STRIPE)],
            out_hbm.at[pl.ds(c * NPAD1 + base, STRIPE)],
        )

    return _deg_sc


def _dis_body(t0_ref, t1_ref, dis_ref):
    d = t0_ref[:, 0:1] + t1_ref[:, 0:1]
    dis_ref[...] = jnp.where(d > 0, lax.rsqrt(jnp.maximum(d, 1e-12)), 0.0)


def _dis_tc(degt):
    return pl.pallas_call(
        _dis_body,
        grid=(NB,),
        in_specs=[
            pl.BlockSpec((MT, SL), lambda i: (i, 0)),
            pl.BlockSpec((MT, SL), lambda i: (NB1 + i, 0)),
        ],
        out_specs=pl.BlockSpec((MT, 1), lambda i: (i, 0)),
        out_shape=jax.ShapeDtypeStruct((NPAD, 1), jnp.float32),
    )(degt, degt)


@functools.lru_cache(maxsize=None)
def _make_spmm(nslab):
    CM = 128
    NROW = E_PAD // (NCORE * NSUB) // CM
    ZR = 16
    NBUF = 2

    @functools.partial(
        pl.kernel,
        out_type=jax.ShapeDtypeStruct((2 * nslab * NPAD1, SL), jnp.float32),
        mesh=_mesh_sc(),
        scratch_types=[
            pltpu.VMEM((NROW, CM), jnp.int32),
            pltpu.VMEM((NROW, CM), jnp.int32),
            pltpu.VMEM((CM, SL), jnp.float32),
            pltpu.VMEM((CM, SL), jnp.float32),
            pltpu.VMEM_SHARED((NPAD1, SL), jnp.float32),
        ] + [pltpu.SemaphoreType.DMA] * 2,
    )
    def spmm(zeros_hbm, z_hbm, gsrc_hbm, sdst_hbm, out_hbm,
             gi_v, si_v, r0, r1, acc_ref,
             g0, g1):
        c = lax.axis_index("c")
        s = lax.axis_index("s")
        bufs = (r0, r1)
        gsem = (g0, g1)
        base = s * STRIPE
        rbase = c * (E_PAD // 2 // CM) + s * NROW
        pltpu.sync_copy(sdst_hbm.at[pl.ds(pl.multiple_of(rbase, 8), NROW)],
                        si_v)

        def gath(j, b):
            pltpu.async_copy(z_hbm.at[gi_v.at[j]], bufs[b], gsem[b])

        def wgath(b):
            pltpu.make_async_copy(z_hbm.at[gi_v.at[0]], bufs[b],
                                  gsem[b]).wait()

        def scat(j, b):
            pltpu.sync_copy(bufs[b], acc_ref.at[si_v.at[j]], add=True)

        for sl in range(nslab):
            pltpu.sync_copy(
                zeros_hbm.at[pl.ds(pl.multiple_of(base, 8), STRIPE)],
                acc_ref.at[pl.ds(pl.multiple_of(base, 8), STRIPE)])
            pltpu.sync_copy(
                gsrc_hbm.at[pl.ds(
                    pl.multiple_of(sl * (E_PAD // CM) + rbase, 8), NROW)],
                gi_v)
            plsc.subcore_barrier()

            for b in range(NBUF):
                gath(b, b)

            def ebody(jj, _):
                for b in range(NBUF):
                    wgath(b)
                    scat(NBUF * jj + b, b)
                    gath(NBUF * jj + NBUF + b, b)
                return 0

            lax.fori_loop(0, NROW // NBUF - 1, ebody, 0)
            for b in range(NBUF):
                wgath(b)
                scat(NROW - NBUF + b, b)
            plsc.subcore_barrier()
            pltpu.sync_copy(
                acc_ref.at[pl.ds(pl.multiple_of(base, 8), STRIPE)],
                out_hbm.at[pl.ds(
                    pl.multiple_of((c * nslab + sl) * NPAD1 + base, 8),
                    STRIPE)],
            )

    return spmm


def _mm_call(As, W0s, b, relu_u=False, W1s=None, dis=None):
    k = len(As)
    cins = [a[1].shape[1] if isinstance(a, tuple) else a.shape[1] for a in As]
    cout = W0s[0].shape[1]
    emit_z = W1s is not None
    cz = W1s[0].shape[1] if emit_z else 0
    nslab = (cz + SL - 1) // SL if emit_z else 0

    in_specs = []
    args = []
    layout = []
    for a in As:
        if isinstance(a, tuple):
            _, u_p, t_p, ns_p = a
            layout.append(ns_p)
            in_specs.append(pl.BlockSpec((MT, u_p.shape[1]),
                                         lambda i: (i, 0)))
            args.append(u_p)
            for cc in range(2):
                for sl2 in range(ns_p):
                    blk = (cc * ns_p + sl2) * NB1
                    in_specs.append(pl.BlockSpec(
                        (MT, SL), lambda i, blk=blk: (blk + i, 0)))
                    args.append(t_p)
        else:
            layout.append(None)
            in_specs.append(pl.BlockSpec((MT, a.shape[1]),
                                         lambda i: (i, 0)))
            args.append(a)
    nA = len(args)
    in_specs += [pl.BlockSpec((c, cout), lambda i: (0, 0)) for c in cins]
    in_specs.append(pl.BlockSpec((1, cout), lambda i: (0, 0)))
    in_specs.append(pl.BlockSpec((MT, 1), lambda i: (i, 0)))
    args += list(W0s) + [b.reshape(1, cout), dis]

    def body(*refs):
        a_refs = refs[:nA]
        w0_refs = refs[nA:nA + k]
        b_ref = refs[nA + k]
        dis_ref = refs[nA + k + 1]
        pos = nA + k + 2
        if emit_z:
            w1_refs = refs[pos:pos + k]
            u_ref, z_ref = refs[pos + k:]
        else:
            u_ref = refs[pos]
        a_vals = []
        ai = 0
        for li in layout:
            if li is None:
                a_vals.append(a_refs[ai][...])
                ai += 1
            else:
                u_p = a_refs[ai][...]
                cu_p = u_p.shape[1]
                slabs = []
                for sl2 in range(li):
                    slabs.append(a_refs[ai + 1 + sl2][...]
                                 + a_refs[ai + 1 + li + sl2][...])
                tx = jnp.concatenate(slabs, axis=1)[:, :cu_p] if li > 1 \
                    else slabs[0][:, :cu_p]
                a_vals.append(jnp.maximum(u_p - dis_ref[...] * tx, 0.0))
                ai += 1 + 2 * li
        u = jnp.broadcast_to(b_ref[...].astype(jnp.float32), (MT, cout))
        for i in range(k):
            u = u + jnp.dot(a_vals[i], w0_refs[i][...],
                            preferred_element_type=jnp.float32)
        if relu_u:
            u = jnp.maximum(u, 0.0)
        u_ref[...] = u
        if emit_z:
            zf = jnp.zeros((MT, cz), jnp.float32)
            for i in range(k):
                zf = zf + jnp.dot(a_vals[i], w1_refs[i][...],
                                  preferred_element_type=jnp.float32)
            zf = zf * dis_ref[...]
            if nslab * SL > cz:
                zf = jnp.concatenate(
                    [zf, jnp.zeros((MT, nslab * SL - cz), jnp.float32)],
                    axis=1)
            for sl in range(nslab):
                z_ref[sl, :, :] = zf[:, sl * SL:(sl + 1) * SL]

    out_shape = [jax.ShapeDtypeStruct((NPAD, cout), jnp.float32)]
    out_specs = [pl.BlockSpec((MT, cout), lambda i: (i, 0))]
    if emit_z:
        in_specs += [pl.BlockSpec((c, cz), lambda i: (0, 0)) for c in cins]
        args += list(W1s)
        out_shape.append(
            jax.ShapeDtypeStruct((nslab, NPAD, SL), jnp.float32))
        out_specs.append(pl.BlockSpec((nslab, MT, SL), lambda i: (0, i, 0)))
    res = pl.pallas_call(
        body,
        grid=(NB,),
        in_specs=in_specs,
        out_specs=out_specs,
        out_shape=out_shape,
    )(*args)
    if emit_z:
        return res[0], res[1].reshape(nslab * NPAD, SL), nslab
    return res[0]


def _comb_call(u, t, dis, nslab, mode, pre=None):
    cu = u.shape[1]

    def body(*refs):
        u_ref = refs[0]
        t_refs = refs[1:1 + 2 * nslab]
        dis_ref = refs[1 + 2 * nslab]
        if pre is None:
            o_ref = refs[2 + 2 * nslab]
        else:
            p_ref = refs[2 + 2 * nslab]
            o_ref = refs[3 + 2 * nslab]
        slabs = []
        for sl in range(nslab):
            slabs.append(t_refs[sl][...] + t_refs[nslab + sl][...])
        tx = jnp.concatenate(slabs, axis=1)[:, :cu] if nslab > 1 \
            else slabs[0][:, :cu]
        v = u_ref[...] - dis_ref[...] * tx
        if mode == "relu":
            o_ref[...] = jnp.maximum(v, 0.0)
        elif mode == "resavg":
            o_ref[...] = (p_ref[...] + jnp.maximum(v, 0.0)) * 0.5
        else:
            o_ref[...] = p_ref[...] + AMP * v

    in_specs = [pl.BlockSpec((MT, cu), lambda i: (i, 0))]
    args = [u]
    for c in range(2):
        for sl in range(nslab):
            blk = (c * nslab + sl) * NB1
            in_specs.append(
                pl.BlockSpec((MT, SL), lambda i, blk=blk: (blk + i, 0)))
            args.append(t)
    in_specs.append(pl.BlockSpec((MT, 1), lambda i: (i, 0)))
    args.append(dis)
    if pre is not None:
        in_specs.append(pl.BlockSpec((MT, cu), lambda i: (i, 0)))
        args.append(pre)
    return pl.pallas_call(
        body,
        grid=(NB,),
        in_specs=in_specs,
        out_specs=pl.BlockSpec((MT, cu), lambda i: (i, 0)),
        out_shape=jax.ShapeDtypeStruct((NPAD, cu), jnp.float32),
    )(*args)


def _zb_body(x_ref, dis_ref, z_ref):
    z_ref[...] = jnp.concatenate(
        [x_ref[...] * dis_ref[...], jnp.zeros((MT, SL - 3), jnp.float32)],
        axis=1)


def _zb(x, dis):
    return pl.pallas_call(
        _zb_body,
        grid=(NB,),
        in_specs=[
            pl.BlockSpec((MT, 3), lambda i: (i, 0)),
            pl.BlockSpec((MT, 1), lambda i: (i, 0)),
        ],
        out_specs=pl.BlockSpec((MT, SL), lambda i: (i, 0)),
        out_shape=jax.ShapeDtypeStruct((NPAD, SL), jnp.float32),
    )(x, dis)


def _s3_body(t0_ref, t1_ref, dis_ref, s_ref):
    s_ref[...] = -dis_ref[...] * (t0_ref[:, :3] + t1_ref[:, :3])


def _s3(t, dis):
    return pl.pallas_call(
        _s3_body,
        grid=(NB,),
        in_specs=[
            pl.BlockSpec((MT, SL), lambda i: (i, 0)),
            pl.BlockSpec((MT, SL), lambda i: (NB1 + i, 0)),
            pl.BlockSpec((MT, 1), lambda i: (i, 0)),
        ],
        out_specs=pl.BlockSpec((MT, 3), lambda i: (i, 0)),
        out_shape=jax.ShapeDtypeStruct((NPAD, 3), jnp.float32),
    )(t, t, dis)


def _vt(vol_cm, ts3=2048):
    c, s3 = vol_cm.shape
    cpad = max(c, SL)
    ts3 = min(ts3, s3)

    def body(v_ref, o_ref):
        vt = v_ref[...].T
        if cpad > c:
            vt = jnp.concatenate(
                [vt, jnp.zeros((ts3, cpad - c), jnp.float32)], axis=1)
        o_ref[...] = vt

    return pl.pallas_call(
        body,
        grid=(s3 // ts3,),
        in_specs=[pl.BlockSpec((c, ts3), lambda j: (0, j))],
        out_specs=pl.BlockSpec((ts3, cpad), lambda j: (j, 0)),
        out_shape=jax.ShapeDtypeStruct((s3, cpad), jnp.float32),
    )(vol_cm)


def _gsidx_call(mesh_x, S):
    def body(m_ref, idx_ref, w_ref):
        m = m_ref[...]
        px = ((m[:, 0] + 1.0) * S - 1.0) * 0.5
        py = ((m[:, 1] + 1.0) * S - 1.0) * 0.5
        pz = ((m[:, 2] + 1.0) * S - 1.0) * 0.5
        x0 = jnp.floor(px); y0 = jnp.floor(py); z0 = jnp.floor(pz)
        xw = px - x0; yw = py - y0; zw = pz - z0
        idxs = []
        wgts = []
        for dz in (0, 1):
            for dy in (0, 1):
                for dx in (0, 1):
                    xi = x0 + dx; yi = y0 + dy; zi = z0 + dz
                    wx = xw if dx else (1.0 - xw)
                    wy = yw if dy else (1.0 - yw)
                    wz = zw if dz else (1.0 - zw)
                    valid = ((xi >= 0) & (xi < S) & (yi >= 0) & (yi < S)
                             & (zi >= 0) & (zi < S))
                    xc = jnp.clip(xi, 0, S - 1).astype(jnp.int32)
                    yc = jnp.clip(yi, 0, S - 1).astype(jnp.int32)
                    zc = jnp.clip(zi, 0, S - 1).astype(jnp.int32)
                    lin = (zc * S + yc) * S + xc
                    w = wx * wy * wz * valid.astype(jnp.float32)
                    idxs.append(lin.reshape(MT, 1))
                    wgts.append(
                        jnp.broadcast_to(w.reshape(MT, 1, 1), (MT, 1, 16)))
        idx_ref[...] = jnp.concatenate(idxs, axis=1)
        w_ref[...] = jnp.concatenate(wgts, axis=1)

    idx8, w3 = pl.pallas_call(
        body,
        grid=(NB,),
        in_specs=[pl.BlockSpec((MT, 3), lambda i: (i, 0))],
        out_specs=[
            pl.BlockSpec((MT, 8), lambda i: (i, 0)),
            pl.BlockSpec((MT, 8, 16), lambda i: (i, 0, 0)),
        ],
        out_shape=[
            jax.ShapeDtypeStruct((NPAD, 8), jnp.int32),
            jax.ShapeDtypeStruct((NPAD, 8, 16), jnp.float32),
        ],
    )(mesh_x)
    return idx8.reshape(NPAD // 16, 128), w3.reshape(NPAD * 128)


@functools.lru_cache(maxsize=None)
def _make_gs(C, CPAD):
    KN = 16
    NPW = NPAD // (NCORE * NSUB)
    NCH = NPW // KN

    @functools.partial(
        pl.kernel,
        out_type=jax.ShapeDtypeStruct((NPAD, C), jnp.float32),
        mesh=_mesh_sc(),
        scratch_types=[
            pltpu.VMEM((NCH + 4, 128), jnp.int32),
            pltpu.VMEM((NPW * 128,), jnp.float32),
            pltpu.VMEM((CK, CPAD), jnp.float32),
            pltpu.VMEM((CK, CPAD), jnp.float32),
            pltpu.VMEM((KN, C), jnp.float32),
            pltpu.SemaphoreType.DMA,
            pltpu.SemaphoreType.DMA,
        ],
    )
    def gs(volf_hbm, idxg_hbm, wgtf_hbm, out_hbm,
           idx_v, wgt_v, rowa, rowb, o_v, ga, gb):
        c = lax.axis_index("c")
        s = lax.axis_index("s")
        wid = s * NCORE + c
        off = 4 * lax.rem(wid, 2)
        astart = pl.multiple_of(wid * NCH - off, 8)
        pltpu.sync_copy(idxg_hbm.at[pl.ds(astart, NCH + 4)], idx_v)
        pltpu.sync_copy(wgtf_hbm.at[pl.ds(wid * NPW * 128, NPW * 128)], wgt_v)

        def gath(g, buf, sem):
            pltpu.async_copy(volf_hbm.at[idx_v.at[g + off]], buf, sem)

        def wait(buf, sem):
            pltpu.make_async_copy(volf_hbm.at[idx_v.at[0]], buf, sem).wait()

        def compute(g, buf):
            wb = g * KN * 128

            def node(n, _):
                ws = [wgt_v[pl.ds(wb + n * 128 + cr * 16, 16)]
                      for cr in range(8)]
                for j in range(C // 16):
                    acc = jnp.zeros((16,), jnp.float32)
                    for cr in range(8):
                        acc = acc + ws[cr] * buf[n * 8 + cr,
                                                 pl.ds(j * 16, 16)]
                    o_v[n, pl.ds(j * 16, 16)] = acc
                return 0

            lax.fori_loop(0, KN, node, 0)
            pltpu.sync_copy(
                o_v, out_hbm.at[pl.ds(wid * NPW + g * KN, KN)])

        gath(0, rowa, ga)
        gath(1, rowb, gb)

        def chunk(jj, _):
            g = 2 * jj
            wait(rowa, ga)
            compute(g, rowa)
            gath(g + 2, rowa, ga)
            wait(rowb, gb)
            compute(g + 1, rowb)
            gath(g + 3, rowb, gb)
            return 0

        lax.fori_loop(0, NCH // 2 - 1, chunk, 0)
        wait(rowa, ga)
        compute(NCH - 2, rowa)
        wait(rowb, gb)
        compute(NCH - 1, rowb)

    return gs


def _conv(x_list, p, dis, gsrc, sdst, zeros_acc, mode, pre=None):
    wslices = []
    o = 0
    for a in x_list:
        ci = a[1].shape[1] if isinstance(a, tuple) else a.shape[1]
        wslices.append((p["W0"][o:o + ci], p["W1"][o:o + ci]))
        o += ci
    u, z, nslab = _mm_call(x_list, [w[0] for w in wslices], p["b"],
                           W1s=[w[1] for w in wslices], dis=dis)
    t = _make_spmm(nslab)(zeros_acc, z, gsrc, sdst)
    if mode == "raw":
        return ("f", u, t, nslab)
    return _comb_call(u, t, dis, nslab, mode, pre=pre)


def kernel(features_0, features_1, features_2, features_3, features_4,
           mesh_template, edge_index, params):
    features = [features_0, features_1, features_2, features_3, features_4]
    p = params

    ei_pad = jnp.pad(edge_index, ((0, 0), (0, E_PAD - E)))
    gsrc, sdst, sdeg = _prep(ei_pad)
    gsrc = gsrc.reshape(MAXSLAB * E_PAD // CK, CK)
    sdst = sdst.reshape(E_PAD // CK, CK)
    sdeg = sdeg.reshape(E_PAD // CK, CK)
    zeros_acc = jnp.zeros((NPAD1, SL), jnp.float32)
    degt = _make_deg()(zeros_acc, sdeg)
    dis = _dis_tc(degt)

    volf = []
    for fi, (S, C) in enumerate(zip(VOL_SIZES, VOL_CH)):
        volf.append(_vt(features[fi].reshape(C, S * S * S)))

    x0 = jnp.pad(mesh_template[0], ((0, NPAD - N_NODES), (0, 0)))

    curr_mesh = x0
    curr_feat = x0
    out_mesh = []
    for i in range(3):
        pi = p[f"init{i}"]
        if i == 0:
            z = _zb(x0, dis)
            t = _make_spmm(1)(zeros_acc, z, gsrc, sdst)
            s = _s3(t, dis)
            curr_feat = _mm_call([x0, s], [pi["W0"], pi["W1"]], pi["b"],
                                 relu_u=True, dis=dis)
        else:
            curr_feat = _conv([curr_feat], pi, dis, gsrc, sdst, zeros_acc, "raw")

        projs = []
        for fl in LEVELS[i]:
            S = VOL_SIZES[fl]
            C = VOL_CH[fl]
            idxg, wgtf = _gsidx_call(curr_mesh, S)
            gsfn = _make_gs(C, max(C, SL))
            projs.append(gsfn(volf[fl], idxg, wgtf))

        y = _conv([curr_feat] + projs, p[f"blk{i}_init"], dis, gsrc, sdst,
                  zeros_acc, "relu")

        for r in range(3):
            t1 = _conv([y], p[f"blk{i}_res{r}_0"], dis, gsrc, sdst,
                       zeros_acc, "raw")
            y = _conv([t1], p[f"blk{i}_res{r}_1"], dis, gsrc, sdst,
                      zeros_acc, "resavg", pre=y)

        curr_mesh = _conv([y], p[f"blk{i}_final"], dis, gsrc, sdst,
                          zeros_acc, "mesh", pre=curr_mesh)
        curr_feat = y
        out_mesh.append(curr_mesh[:N_NODES][None])
    return tuple(out_mesh)


# speedup vs baseline: 1.0275x; 1.0275x over previous
import functools

import jax
import jax.numpy as jnp
from jax import lax
from jax.experimental import pallas as pl
from jax.experimental.pallas import tpu as pltpu
from jax.experimental.pallas import tpu_sc as plsc

N_NODES = 10000
NPAD = 10240
NPAD1 = 10496
DUMP = NPAD
E = 160000
E_PAD = 163840
NSUB = 16
NCORE = 2
CK = 128
SL = 128
MT = 256
NB = NPAD // MT
NB1 = NPAD1 // MT
STRIPE = NPAD1 // NSUB
AMP = 0.1
LEVELS = [[3, 4], [1, 2], [0, 1]]
VOL_SIZES = (64, 32, 16, 8, 4)
VOL_CH = (16, 32, 64, 128, 256)
MAXSLAB = 3


@functools.lru_cache(maxsize=None)
def _mesh_sc():
    return plsc.VectorSubcoreMesh(core_axis_name="c", subcore_axis_name="s")


def _prep_body(ei_ref, gsrc_ref, sdst_ref, sdeg_ref):
    src = ei_ref[0, :]
    dst = ei_ref[1, :]
    nonself = src != dst
    sdst_ref[...] = jnp.where(nonself, dst, DUMP)
    sdeg_ref[...] = jnp.where(nonself, src, DUMP)
    gsrc_ref[...] = jnp.concatenate(
        [src + k * NPAD for k in range(MAXSLAB)], axis=0)


def _prep(ei_pad):
    return pl.pallas_call(
        _prep_body,
        out_shape=(
            jax.ShapeDtypeStruct((MAXSLAB * E_PAD,), jnp.int32),
            jax.ShapeDtypeStruct((E_PAD,), jnp.int32),
            jax.ShapeDtypeStruct((E_PAD,), jnp.int32),
        ),
    )(ei_pad)


@functools.lru_cache(maxsize=None)
def _make_deg():
    @functools.partial(
        pl.kernel,
        out_type=jax.ShapeDtypeStruct((2 * NPAD1, SL), jnp.float32),
        mesh=_mesh_sc(),
        scratch_types=[
            pltpu.VMEM((E_PAD // (NCORE * NSUB) // CK, CK), jnp.int32),
            pltpu.VMEM((CK, SL), jnp.float32),
            pltpu.VMEM_SHARED((NPAD1, SL), jnp.float32),
            pltpu.SemaphoreType.DMA,
            pltpu.SemaphoreType.DMA,
            pltpu.SemaphoreType.DMA,
            pltpu.SemaphoreType.DMA,
        ],
    )
    def _deg_sc(zeros_hbm, sdeg_hbm, out_hbm, si_v, ones_v, acc_ref,
                s0, s1, s2, s3):
        c = lax.axis_index("c")
        s = lax.axis_index("s")

        def obody(r, _):
            for j in range(SL // 16):
                ones_v[r, pl.ds(j * 16, 16)] = jnp.ones((16,), jnp.float32)
            return 0

        lax.fori_loop(0, CK, obody, 0)
        base = s * STRIPE
        nrow = E_PAD // (NCORE * NSUB) // CK
        rbase = c * (E_PAD // 2 // CK) + s * nrow
        pltpu.sync_copy(sdeg_hbm.at[pl.ds(rbase, nrow)], si_v)
        pltpu.sync_copy(
            zeros_hbm.at[pl.ds(pl.multiple_of(base, 8), STRIPE)],
            acc_ref.at[pl.ds(pl.multiple_of(base, 8), STRIPE)])
        plsc.subcore_barrier()
        sems = (s0, s1, s2, s3)

        def ebody(k, _):
            for b in range(4):
                pltpu.async_copy(ones_v, acc_ref.at[si_v.at[4 * k + b]],
                                 sems[b], add=True)
            for b in range(4):
                pltpu.make_async_copy(out_hbm.at[pl.ds(0, CK)], ones_v,
                                      sems[b]).wait()
            return 0

        lax.fori_loop(0, E_PAD // (NCORE * NSUB) // CK // 4, ebody, 0)
        plsc.subcore_barrier()
        pltpu.sync_copy(
            acc_ref.at[pl.ds(base, STRIPE)],
            out_hbm.at[pl.ds(c * NPAD1 + base, STRIPE)],
        )

    return _deg_sc


def _dis_body(t0_ref, t1_ref, dis_ref):
    d = t0_ref[:, 0:1] + t1_ref[:, 0:1]
    dis_ref[...] = jnp.where(d > 0, lax.rsqrt(jnp.maximum(d, 1e-12)), 0.0)


def _dis_tc(degt):
    return pl.pallas_call(
        _dis_body,
        grid=(NB,),
        in_specs=[
            pl.BlockSpec((MT, SL), lambda i: (i, 0)),
            pl.BlockSpec((MT, SL), lambda i: (NB1 + i, 0)),
        ],
        out_specs=pl.BlockSpec((MT, 1), lambda i: (i, 0)),
        out_shape=jax.ShapeDtypeStruct((NPAD, 1), jnp.float32),
    )(degt, degt)


@functools.lru_cache(maxsize=None)
def _make_spmm(nslab):
    CM = 128
    NROW = E_PAD // (NCORE * NSUB) // CM
    ZR = 16
    NBUF = 2

    @functools.partial(
        pl.kernel,
        out_type=jax.ShapeDtypeStruct((2 * nslab * NPAD1, SL), jnp.float32),
        mesh=_mesh_sc(),
        scratch_types=[
            pltpu.VMEM((NROW, CM), jnp.int32),
            pltpu.VMEM((NROW, CM), jnp.int32),
            pltpu.VMEM((CM, SL), jnp.float32),
            pltpu.VMEM((CM, SL), jnp.float32),
            pltpu.VMEM_SHARED((NPAD1, SL), jnp.float32),
        ] + [pltpu.SemaphoreType.DMA] * 2,
    )
    def spmm(zeros_hbm, z_hbm, gsrc_hbm, sdst_hbm, out_hbm,
             gi_v, si_v, r0, r1, acc_ref,
             g0, g1):
        c = lax.axis_index("c")
        s = lax.axis_index("s")
        bufs = (r0, r1)
        gsem = (g0, g1)
        base = s * STRIPE
        rbase = c * (E_PAD // 2 // CM) + s * NROW
        pltpu.sync_copy(sdst_hbm.at[pl.ds(pl.multiple_of(rbase, 8), NROW)],
                        si_v)

        def gath(j, b):
            pltpu.async_copy(z_hbm.at[gi_v.at[j]], bufs[b], gsem[b])

        def wgath(b):
            pltpu.make_async_copy(z_hbm.at[gi_v.at[0]], bufs[b],
                                  gsem[b]).wait()

        def scat(j, b):
            pltpu.sync_copy(bufs[b], acc_ref.at[si_v.at[j]], add=True)

        for sl in range(nslab):
            pltpu.sync_copy(
                zeros_hbm.at[pl.ds(pl.multiple_of(base, 8), STRIPE)],
                acc_ref.at[pl.ds(pl.multiple_of(base, 8), STRIPE)])
            pltpu.sync_copy(
                gsrc_hbm.at[pl.ds(
                    pl.multiple_of(sl * (E_PAD // CM) + rbase, 8), NROW)],
                gi_v)
            plsc.subcore_barrier()

            for b in range(NBUF):
                gath(b, b)

            def ebody(jj, _):
                for b in range(NBUF):
                    wgath(b)
                    scat(NBUF * jj + b, b)
                    gath(NBUF * jj + NBUF + b, b)
                return 0

            lax.fori_loop(0, NROW // NBUF - 1, ebody, 0)
            for b in range(NBUF):
                wgath(b)
                scat(NROW - NBUF + b, b)
            plsc.subcore_barrier()
            pltpu.sync_copy(
                acc_ref.at[pl.ds(pl.multiple_of(base, 8), STRIPE)],
                out_hbm.at[pl.ds(
                    pl.multiple_of((c * nslab + sl) * NPAD1 + base, 8),
                    STRIPE)],
            )

    return spmm


def _mm_call(As, W0s, b, relu_u=False, W1s=None, dis=None):
    k = len(As)
    cins = [a[1].shape[1] if isinstance(a, tuple) else a.shape[1] for a in As]
    cout = W0s[0].shape[1]
    emit_z = W1s is not None
    cz = W1s[0].shape[1] if emit_z else 0
    nslab = (cz + SL - 1) // SL if emit_z else 0

    in_specs = []
    args = []
    layout = []
    for a in As:
        if isinstance(a, tuple):
            _, u_p, t_p, ns_p = a
            layout.append(ns_p)
            in_specs.append(pl.BlockSpec((MT, u_p.shape[1]),
                                         lambda i: (i, 0)))
            args.append(u_p)
            for cc in range(2):
                for sl2 in range(ns_p):
                    blk = (cc * ns_p + sl2) * NB1
                    in_specs.append(pl.BlockSpec(
                        (MT, SL), lambda i, blk=blk: (blk + i, 0)))
                    args.append(t_p)
        else:
            layout.append(None)
            in_specs.append(pl.BlockSpec((MT, a.shape[1]),
                                         lambda i: (i, 0)))
            args.append(a)
    nA = len(args)
    in_specs += [pl.BlockSpec((c, cout), lambda i: (0, 0)) for c in cins]
    in_specs.append(pl.BlockSpec((1, cout), lambda i: (0, 0)))
    in_specs.append(pl.BlockSpec((MT, 1), lambda i: (i, 0)))
    args += list(W0s) + [b.reshape(1, cout), dis]

    def body(*refs):
        a_refs = refs[:nA]
        w0_refs = refs[nA:nA + k]
        b_ref = refs[nA + k]
        dis_ref = refs[nA + k + 1]
        pos = nA + k + 2
        if emit_z:
            w1_refs = refs[pos:pos + k]
            u_ref, z_ref = refs[pos + k:]
        else:
            u_ref = refs[pos]
        a_vals = []
        ai = 0
        for li in layout:
            if li is None:
                a_vals.append(a_refs[ai][...])
                ai += 1
            else:
                u_p = a_refs[ai][...]
                cu_p = u_p.shape[1]
                slabs = []
                for sl2 in range(li):
                    slabs.append(a_refs[ai + 1 + sl2][...]
                                 + a_refs[ai + 1 + li + sl2][...])
                tx = jnp.concatenate(slabs, axis=1)[:, :cu_p] if li > 1 \
                    else slabs[0][:, :cu_p]
                a_vals.append(jnp.maximum(u_p - dis_ref[...] * tx, 0.0))
                ai += 1 + 2 * li
        u = jnp.broadcast_to(b_ref[...].astype(jnp.float32), (MT, cout))
        for i in range(k):
            u = u + jnp.dot(a_vals[i], w0_refs[i][...],
                            preferred_element_type=jnp.float32)
        if relu_u:
            u = jnp.maximum(u, 0.0)
        u_ref[...] = u
        if emit_z:
            zf = jnp.zeros((MT, cz), jnp.float32)
            for i in range(k):
                zf = zf + jnp.dot(a_vals[i], w1_refs[i][...],
                                  preferred_element_type=jnp.float32)
            zf = zf * dis_ref[...]
            if nslab * SL > cz:
                zf = jnp.concatenate(
                    [zf, jnp.zeros((MT, nslab * SL - cz), jnp.float32)],
                    axis=1)
            for sl in range(nslab):
                z_ref[sl, :, :] = zf[:, sl * SL:(sl + 1) * SL]

    out_shape = [jax.ShapeDtypeStruct((NPAD, cout), jnp.float32)]
    out_specs = [pl.BlockSpec((MT, cout), lambda i: (i, 0))]
    if emit_z:
        in_specs += [pl.BlockSpec((c, cz), lambda i: (0, 0)) for c in cins]
        args += list(W1s)
        out_shape.append(
            jax.ShapeDtypeStruct((nslab, NPAD, SL), jnp.float32))
        out_specs.append(pl.BlockSpec((nslab, MT, SL), lambda i: (0, i, 0)))
    res = pl.pallas_call(
        body,
        grid=(NB,),
        in_specs=in_specs,
        out_specs=out_specs,
        out_shape=out_shape,
    )(*args)
    if emit_z:
        return res[0], res[1].reshape(nslab * NPAD, SL), nslab
    return res[0]


def _comb_call(u, t, dis, nslab, mode, pre=None):
    cu = u.shape[1]

    def body(*refs):
        u_ref = refs[0]
        t_refs = refs[1:1 + 2 * nslab]
        dis_ref = refs[1 + 2 * nslab]
        if pre is None:
            o_ref = refs[2 + 2 * nslab]
        else:
            p_ref = refs[2 + 2 * nslab]
            o_ref = refs[3 + 2 * nslab]
        slabs = []
        for sl in range(nslab):
            slabs.append(t_refs[sl][...] + t_refs[nslab + sl][...])
        tx = jnp.concatenate(slabs, axis=1)[:, :cu] if nslab > 1 \
            else slabs[0][:, :cu]
        v = u_ref[...] - dis_ref[...] * tx
        if mode == "relu":
            o_ref[...] = jnp.maximum(v, 0.0)
        elif mode == "resavg":
            o_ref[...] = (p_ref[...] + jnp.maximum(v, 0.0)) * 0.5
        else:
            o_ref[...] = p_ref[...] + AMP * v

    in_specs = [pl.BlockSpec((MT, cu), lambda i: (i, 0))]
    args = [u]
    for c in range(2):
        for sl in range(nslab):
            blk = (c * nslab + sl) * NB1
            in_specs.append(
                pl.BlockSpec((MT, SL), lambda i, blk=blk: (blk + i, 0)))
            args.append(t)
    in_specs.append(pl.BlockSpec((MT, 1), lambda i: (i, 0)))
    args.append(dis)
    if pre is not None:
        in_specs.append(pl.BlockSpec((MT, cu), lambda i: (i, 0)))
        args.append(pre)
    return pl.pallas_call(
        body,
        grid=(NB,),
        in_specs=in_specs,
        out_specs=pl.BlockSpec((MT, cu), lambda i: (i, 0)),
        out_shape=jax.ShapeDtypeStruct((NPAD, cu), jnp.float32),
    )(*args)


def _zb_body(x_ref, dis_ref, z_ref):
    z_ref[...] = jnp.concatenate(
        [x_ref[...] * dis_ref[...], jnp.zeros((MT, SL - 3), jnp.float32)],
        axis=1)


def _zb(x, dis):
    return pl.pallas_call(
        _zb_body,
        grid=(NB,),
        in_specs=[
            pl.BlockSpec((MT, 3), lambda i: (i, 0)),
            pl.BlockSpec((MT, 1), lambda i: (i, 0)),
        ],
        out_specs=pl.BlockSpec((MT, SL), lambda i: (i, 0)),
        out_shape=jax.ShapeDtypeStruct((NPAD, SL), jnp.float32),
    )(x, dis)


def _s3_body(t0_ref, t1_ref, dis_ref, s_ref):
    s_ref[...] = -dis_ref[...] * (t0_ref[:, :3] + t1_ref[:, :3])


def _s3(t, dis):
    return pl.pallas_call(
        _s3_body,
        grid=(NB,),
        in_specs=[
            pl.BlockSpec((MT, SL), lambda i: (i, 0)),
            pl.BlockSpec((MT, SL), lambda i: (NB1 + i, 0)),
            pl.BlockSpec((MT, 1), lambda i: (i, 0)),
        ],
        out_specs=pl.BlockSpec((MT, 3), lambda i: (i, 0)),
        out_shape=jax.ShapeDtypeStruct((NPAD, 3), jnp.float32),
    )(t, t, dis)


def _vt(vol_cm, ts3=2048):
    c, s3 = vol_cm.shape
    cpad = max(c, SL)
    ts3 = min(ts3, s3)

    def body(v_ref, o_ref):
        vt = v_ref[...].T
        if cpad > c:
            vt = jnp.concatenate(
                [vt, jnp.zeros((ts3, cpad - c), jnp.float32)], axis=1)
        o_ref[...] = vt

    return pl.pallas_call(
        body,
        grid=(s3 // ts3,),
        in_specs=[pl.BlockSpec((c, ts3), lambda j: (0, j))],
        out_specs=pl.BlockSpec((ts3, cpad), lambda j: (j, 0)),
        out_shape=jax.ShapeDtypeStruct((s3, cpad), jnp.float32),
    )(vol_cm)


def _gsidx_call(mesh_x, S):
    def body(m_ref, idx_ref, w_ref):
        m = m_ref[...]
        px = ((m[:, 0] + 1.0) * S - 1.0) * 0.5
        py = ((m[:, 1] + 1.0) * S - 1.0) * 0.5
        pz = ((m[:, 2] + 1.0) * S - 1.0) * 0.5
        x0 = jnp.floor(px); y0 = jnp.floor(py); z0 = jnp.floor(pz)
        xw = px - x0; yw = py - y0; zw = pz - z0
        idxs = []
        wgts = []
        for dz in (0, 1):
            for dy in (0, 1):
                for dx in (0, 1):
                    xi = x0 + dx; yi = y0 + dy; zi = z0 + dz
                    wx = xw if dx else (1.0 - xw)
                    wy = yw if dy else (1.0 - yw)
                    wz = zw if dz else (1.0 - zw)
                    valid = ((xi >= 0) & (xi < S) & (yi >= 0) & (yi < S)
                             & (zi >= 0) & (zi < S))
                    xc = jnp.clip(xi, 0, S - 1).astype(jnp.int32)
                    yc = jnp.clip(yi, 0, S - 1).astype(jnp.int32)
                    zc = jnp.clip(zi, 0, S - 1).astype(jnp.int32)
                    lin = (zc * S + yc) * S + xc
                    w = wx * wy * wz * valid.astype(jnp.float32)
                    idxs.append(lin.reshape(MT, 1))
                    wgts.append(
                        jnp.broadcast_to(w.reshape(MT, 1, 1), (MT, 1, 16)))
        idx_ref[...] = jnp.concatenate(idxs, axis=1)
        w_ref[...] = jnp.concatenate(wgts, axis=1)

    idx8, w3 = pl.pallas_call(
        body,
        grid=(NB,),
        in_specs=[pl.BlockSpec((MT, 3), lambda i: (i, 0))],
        out_specs=[
            pl.BlockSpec((MT, 8), lambda i: (i, 0)),
            pl.BlockSpec((MT, 8, 16), lambda i: (i, 0, 0)),
        ],
        out_shape=[
            jax.ShapeDtypeStruct((NPAD, 8), jnp.int32),
            jax.ShapeDtypeStruct((NPAD, 8, 16), jnp.float32),
        ],
    )(mesh_x)
    return idx8.reshape(NPAD // 16, 128), w3.reshape(NPAD * 128)


def _proj_tc(mesh_x, volf, Sz):
    s3, C = volf.shape

    def body(m_ref, v_ref, o_ref):
        m = m_ref[...]
        px = ((m[:, 0] + 1.0) * Sz - 1.0) * 0.5
        py = ((m[:, 1] + 1.0) * Sz - 1.0) * 0.5
        pz = ((m[:, 2] + 1.0) * Sz - 1.0) * 0.5
        x0 = jnp.floor(px); y0 = jnp.floor(py); z0 = jnp.floor(pz)
        xw = px - x0; yw = py - y0; zw = pz - z0
        cols = jax.lax.broadcasted_iota(jnp.int32, (MT, s3), 1)
        sel = jnp.zeros((MT, s3), jnp.float32)
        for dz in (0, 1):
            for dy in (0, 1):
                for dx in (0, 1):
                    xi = x0 + dx; yi = y0 + dy; zi = z0 + dz
                    wx = xw if dx else (1.0 - xw)
                    wy = yw if dy else (1.0 - yw)
                    wz = zw if dz else (1.0 - zw)
                    valid = ((xi >= 0) & (xi < Sz) & (yi >= 0) & (yi < Sz)
                             & (zi >= 0) & (zi < Sz))
                    xc = jnp.clip(xi, 0, Sz - 1).astype(jnp.int32)
                    yc = jnp.clip(yi, 0, Sz - 1).astype(jnp.int32)
                    zc = jnp.clip(zi, 0, Sz - 1).astype(jnp.int32)
                    lin = (zc * Sz + yc) * Sz + xc
                    w = wx * wy * wz * valid.astype(jnp.float32)
                    sel = sel + jnp.where(cols == lin[:, None],
                                          w[:, None], 0.0)
        o_ref[...] = jnp.dot(sel, v_ref[...],
                             preferred_element_type=jnp.float32)

    return pl.pallas_call(
        body,
        grid=(NB,),
        in_specs=[
            pl.BlockSpec((MT, 3), lambda i: (i, 0)),
            pl.BlockSpec((s3, C), lambda i: (0, 0)),
        ],
        out_specs=pl.BlockSpec((MT, C), lambda i: (i, 0)),
        out_shape=jax.ShapeDtypeStruct((NPAD, C), jnp.float32),
    )(mesh_x, volf)


@functools.lru_cache(maxsize=None)
def _make_gs(C, CPAD):
    KN = 16
    NPW = NPAD // (NCORE * NSUB)
    NCH = NPW // KN

    @functools.partial(
        pl.kernel,
        out_type=jax.ShapeDtypeStruct((NPAD, C), jnp.float32),
        mesh=_mesh_sc(),
        scratch_types=[
            pltpu.VMEM((NCH + 4, 128), jnp.int32),
            pltpu.VMEM((NPW * 128,), jnp.float32),
            pltpu.VMEM((CK, CPAD), jnp.float32),
            pltpu.VMEM((CK, CPAD), jnp.float32),
            pltpu.VMEM((KN, C), jnp.float32),
            pltpu.SemaphoreType.DMA,
            pltpu.SemaphoreType.DMA,
        ],
    )
    def gs(volf_hbm, idxg_hbm, wgtf_hbm, out_hbm,
           idx_v, wgt_v, rowa, rowb, o_v, ga, gb):
        c = lax.axis_index("c")
        s = lax.axis_index("s")
        wid = s * NCORE + c
        off = 4 * lax.rem(wid, 2)
        astart = pl.multiple_of(wid * NCH - off, 8)
        pltpu.sync_copy(idxg_hbm.at[pl.ds(astart, NCH + 4)], idx_v)
        pltpu.sync_copy(wgtf_hbm.at[pl.ds(wid * NPW * 128, NPW * 128)], wgt_v)

        def gath(g, buf, sem):
            pltpu.async_copy(volf_hbm.at[idx_v.at[g + off]], buf, sem)

        def wait(buf, sem):
            pltpu.make_async_copy(volf_hbm.at[idx_v.at[0]], buf, sem).wait()

        def compute(g, buf):
            wb = g * KN * 128

            def node(n, _):
                ws = [wgt_v[pl.ds(wb + n * 128 + cr * 16, 16)]
                      for cr in range(8)]
                for j in range(C // 16):
                    acc = jnp.zeros((16,), jnp.float32)
                    for cr in range(8):
                        acc = acc + ws[cr] * buf[n * 8 + cr,
                                                 pl.ds(j * 16, 16)]
                    o_v[n, pl.ds(j * 16, 16)] = acc
                return 0

            lax.fori_loop(0, KN, node, 0)
            pltpu.sync_copy(
                o_v, out_hbm.at[pl.ds(wid * NPW + g * KN, KN)])

        gath(0, rowa, ga)
        gath(1, rowb, gb)

        def chunk(jj, _):
            g = 2 * jj
            wait(rowa, ga)
            compute(g, rowa)
            gath(g + 2, rowa, ga)
            wait(rowb, gb)
            compute(g + 1, rowb)
            gath(g + 3, rowb, gb)
            return 0

        lax.fori_loop(0, NCH // 2 - 1, chunk, 0)
        wait(rowa, ga)
        compute(NCH - 2, rowa)
        wait(rowb, gb)
        compute(NCH - 1, rowb)

    return gs


def _conv(x_list, p, dis, gsrc, sdst, zeros_acc, mode, pre=None):
    wslices = []
    o = 0
    for a in x_list:
        ci = a[1].shape[1] if isinstance(a, tuple) else a.shape[1]
        wslices.append((p["W0"][o:o + ci], p["W1"][o:o + ci]))
        o += ci
    u, z, nslab = _mm_call(x_list, [w[0] for w in wslices], p["b"],
                           W1s=[w[1] for w in wslices], dis=dis)
    t = _make_spmm(nslab)(zeros_acc, z, gsrc, sdst)
    if mode == "raw":
        return ("f", u, t, nslab)
    return _comb_call(u, t, dis, nslab, mode, pre=pre)


def kernel(features_0, features_1, features_2, features_3, features_4,
           mesh_template, edge_index, params):
    features = [features_0, features_1, features_2, features_3, features_4]
    p = params

    ei_pad = jnp.pad(edge_index, ((0, 0), (0, E_PAD - E)))
    gsrc, sdst, sdeg = _prep(ei_pad)
    gsrc = gsrc.reshape(MAXSLAB * E_PAD // CK, CK)
    sdst = sdst.reshape(E_PAD // CK, CK)
    sdeg = sdeg.reshape(E_PAD // CK, CK)
    zeros_acc = jnp.zeros((NPAD1, SL), jnp.float32)
    degt = _make_deg()(zeros_acc, sdeg)
    dis = _dis_tc(degt)

    volf = []
    for fi, (S, C) in enumerate(zip(VOL_SIZES, VOL_CH)):
        volf.append(_vt(features[fi].reshape(C, S * S * S)))

    x0 = jnp.pad(mesh_template[0], ((0, NPAD - N_NODES), (0, 0)))

    curr_mesh = x0
    curr_feat = x0
    out_mesh = []
    for i in range(3):
        pi = p[f"init{i}"]
        if i == 0:
            z = _zb(x0, dis)
            t = _make_spmm(1)(zeros_acc, z, gsrc, sdst)
            s = _s3(t, dis)
            curr_feat = _mm_call([x0, s], [pi["W0"], pi["W1"]], pi["b"],
                                 relu_u=True, dis=dis)
        else:
            curr_feat = _conv([curr_feat], pi, dis, gsrc, sdst, zeros_acc, "raw")

        projs = []
        for fl in LEVELS[i]:
            S = VOL_SIZES[fl]
            C = VOL_CH[fl]
            if S * S * S <= 512:
                projs.append(_proj_tc(curr_mesh, volf[fl], S))
            else:
                idxg, wgtf = _gsidx_call(curr_mesh, S)
                gsfn = _make_gs(C, max(C, SL))
                projs.append(gsfn(volf[fl], idxg, wgtf))

        y = _conv([curr_feat] + projs, p[f"blk{i}_init"], dis, gsrc, sdst,
                  zeros_acc, "relu")

        for r in range(3):
            t1 = _conv([y], p[f"blk{i}_res{r}_0"], dis, gsrc, sdst,
                       zeros_acc, "raw")
            y = _conv([t1], p[f"blk{i}_res{r}_1"], dis, gsrc, sdst,
                      zeros_acc, "resavg", pre=y)

        curr_mesh = _conv([y], p[f"blk{i}_final"], dis, gsrc, sdst,
                          zeros_acc, "mesh", pre=curr_mesh)
        curr_feat = y
        out_mesh.append(curr_mesh[:N_NODES][None])
    return tuple(out_mesh)
